# baseline (speedup 1.0000x reference)
import functools

import jax
import jax.numpy as jnp
from jax import lax
from jax.experimental import pallas as pl
from jax.experimental.pallas import tpu as pltpu
from jax.experimental.pallas import tpu_sc as plsc

N = 10000
NFEAT = 128
NHID = 64
E = 320000

NC = 2
NS = 16
EL = 96
KB = 8
EG = KB * EL
G0 = 21
G1 = 6
GT = G0 + G1
NGRP = NS * GT
NGRP_ALLOC = NGRP + G0
EPAD = NGRP * EG
NPAD = N + 112
RPT = NPAD // NS


def _segsum_body(src_hbm, dst_hbm, vals_hbm, zeros_hbm, out_hbm,
                 src_v, dst_v, rows_v, acc, sem_g):
    cid = lax.axis_index("c")
    sid = lax.axis_index("s")
    base = sid * GT + cid * G0
    n_groups = jnp.where(cid == 0, G0, G1)

    pltpu.sync_copy(zeros_hbm.at[pl.ds(sid * RPT, RPT)],
                    acc.at[pl.ds(sid * RPT, RPT)])
    pltpu.sync_copy(src_hbm.at[pl.ds(base, G0)], src_v)
    pltpu.sync_copy(dst_hbm.at[pl.ds(base, G0)], dst_v)
    plsc.subcore_barrier()

    def body(g, carry):
        gats = [
            pltpu.async_copy(vals_hbm.at[src_v.at[g, b]],
                             rows_v.at[b], sem_g)
            for b in range(KB)
        ]
        for c in gats:
            c.wait()
        for b in range(KB):
            pltpu.sync_copy(rows_v.at[b],
                            acc.at[dst_v.at[g, b]], add=True)
        return carry

    lax.fori_loop(0, n_groups, body, 0)

    plsc.subcore_barrier()
    pltpu.sync_copy(acc.at[pl.ds(sid * RPT, RPT)],
                    out_hbm.at[cid, pl.ds(sid * RPT, RPT)])


_segsum = functools.partial(
    pl.kernel,
    mesh=plsc.VectorSubcoreMesh(core_axis_name="c", subcore_axis_name="s"),
    out_type=jax.ShapeDtypeStruct((NC, NPAD, NHID), jnp.float32),
    scratch_types=[
        pltpu.VMEM((G0, KB, EL), jnp.int32),
        pltpu.VMEM((G0, KB, EL), jnp.int32),
        pltpu.VMEM((KB, EL, NHID), jnp.float32),
        pltpu.VMEM_SHARED((NPAD, NHID), jnp.float32),
        pltpu.SemaphoreType.DMA,
    ],
    compiler_params=pltpu.CompilerParams(use_tc_tiling_on_sc=False),
)(_segsum_body)


def _mm_body(a_ref, w_ref, o_ref):
    o_ref[...] = jnp.dot(a_ref[...], w_ref[...],
                         preferred_element_type=jnp.float32)


def _proj(a, w):
    return pl.pallas_call(
        _mm_body,
        out_shape=jax.ShapeDtypeStruct((a.shape[0], w.shape[1]), jnp.float32),
    )(a, w)


def _comb_relu_mm_body(p_ref, b_ref, w_ref, o_ref):
    x = jnp.maximum(p_ref[0] + p_ref[1] + b_ref[...], 0.0)
    o_ref[...] = jnp.dot(x, w_ref[...], preferred_element_type=jnp.float32)


def _comb_relu_mm(p, b, w):
    return pl.pallas_call(
        _comb_relu_mm_body,
        out_shape=jax.ShapeDtypeStruct((p.shape[1], w.shape[1]), jnp.float32),
    )(p, b.reshape(1, -1), w)


def _comb_relu_body(p_ref, b_ref, o_ref):
    o_ref[...] = jnp.maximum(p_ref[0] + p_ref[1] + b_ref[...], 0.0)


def _comb_relu(p, b):
    return pl.pallas_call(
        _comb_relu_body,
        out_shape=jax.ShapeDtypeStruct((p.shape[1], p.shape[2]), jnp.float32),
    )(p, b.reshape(1, -1))


def _comb_mm2_body(p_ref, wa_ref, ba_ref, ws_ref, bs_ref, xa_ref, s_ref):
    c = p_ref[0] + p_ref[1]
    xa_ref[...] = jnp.maximum(
        jnp.dot(c, wa_ref[...], preferred_element_type=jnp.float32)
        + ba_ref[...], 0.0)
    s_ref[...] = jnp.maximum(
        jnp.dot(c, ws_ref[...], preferred_element_type=jnp.float32)
        + bs_ref[...], 0.0)


def _comb_mm2(p, wa, ba, ws, bs):
    return pl.pallas_call(
        _comb_mm2_body,
        out_shape=(
            jax.ShapeDtypeStruct((p.shape[1], wa.shape[1]), jnp.float32),
            jax.ShapeDtypeStruct((p.shape[1], ws.shape[1]), jnp.float32),
        ),
    )(p, wa, ba.reshape(1, -1), ws, bs.reshape(1, -1))


def _comb_mm_relu_body(p_ref, w_ref, b_ref, o_ref):
    o_ref[...] = jnp.maximum(
        jnp.dot(p_ref[0] + p_ref[1], w_ref[...],
                preferred_element_type=jnp.float32) + b_ref[...], 0.0)


def _comb_mm_relu(p, w, b):
    return pl.pallas_call(
        _comb_mm_relu_body,
        out_shape=jax.ShapeDtypeStruct((p.shape[1], w.shape[1]), jnp.float32),
    )(p, w, b.reshape(1, -1))


_BM = 128


def _outer_body(a_ref, b_ref, o_ref):
    o_ref[...] = lax.dot_general(
        a_ref[...], b_ref[...], (((1,), (1,)), ((), ())),
        preferred_element_type=jnp.float32)


def _outer(s):
    n = s.shape[0]
    return pl.pallas_call(
        _outer_body,
        grid=(pl.cdiv(n, _BM),),
        in_specs=[
            pl.BlockSpec((_BM, NHID), lambda i: (i, 0)),
            pl.BlockSpec((n, NHID), lambda i: (0, 0)),
        ],
        out_specs=pl.BlockSpec((_BM, n), lambda i: (i, 0)),
        out_shape=jax.ShapeDtypeStruct((n, n), jnp.float32),
    )(s, s)


def kernel(h, edge_index, W_e1, b_e1, W_e2, b_e2, W_a1, b_a1, W_a2, b_a2,
           W_s1, b_s1):
    src = edge_index[0].astype(jnp.int32)
    dst = edge_index[1].astype(jnp.int32)
    pad = EPAD - E
    extra = (NGRP_ALLOC - NGRP) * EG
    src2d = jnp.concatenate(
        [src, jnp.zeros((pad + extra,), jnp.int32)]).reshape(
            NGRP_ALLOC, KB, EL)
    dst2d = jnp.concatenate(
        [dst, jnp.full((pad,), N, jnp.int32),
         jnp.zeros((extra,), jnp.int32)]).reshape(NGRP_ALLOC, KB, EL)
    zeros = jnp.zeros((NPAD, NHID), jnp.float32)

    def segsum(vals):
        out = _segsum(src2d, dst2d, vals, zeros)
        return out[:, :N, :]

    m1 = _proj(h, W_e1)
    p = segsum(m1)
    x1m = _comb_relu_mm(p, b_e1, W_e2)
    q = segsum(x1m)
    x2 = _comb_relu(q, b_e2)
    r = segsum(x2)
    xa, s = _comb_mm2(r, W_a1, b_a1, W_s1, b_s1)
    t = segsum(xa)
    x_hat = _comb_mm_relu(t, W_a2, b_a2)
    struct = _outer(s)
    return (struct, x_hat)


# speedup vs baseline: 1.4750x; 1.4750x over previous
import functools

import jax
import jax.numpy as jnp
from jax import lax
from jax.experimental import pallas as pl
from jax.experimental.pallas import tpu as pltpu
from jax.experimental.pallas import tpu_sc as plsc

N = 10000
NFEAT = 128
NHID = 64
E = 320000

NC = 2
NS = 16
EL = 128
KB = 8
EG = KB * EL
SG = 4
G0 = 15
G1 = 5
GT = G0 + G1
NGRP = NS * GT
NGRP_ALLOC = NGRP + 8
EPAD = NGRP * EG
NPAD = N + 112
RPT = NPAD // NS


def _segsum_body(src_hbm, dst_hbm, vals_hbm, zeros_hbm, out_hbm,
                 src_v, dst_v, rows_v, acc, sem_g, sem_i):
    cid = lax.axis_index("c")
    sid = lax.axis_index("s")

    pltpu.sync_copy(zeros_hbm.at[pl.ds(sid * RPT, RPT)],
                    acc.at[pl.ds(sid * RPT, RPT)])
    plsc.subcore_barrier()

    def run(n_groups, off):
        base = sid * GT + off
        n_stages = (n_groups + SG - 1) // SG

        def load_stage(s):
            slot = s & 1
            return [
                pltpu.async_copy(src_hbm.at[pl.ds(base + s * SG, SG)],
                                 src_v.at[slot], sem_i),
                pltpu.async_copy(dst_hbm.at[pl.ds(base + s * SG, SG)],
                                 dst_v.at[slot], sem_i),
            ]

        pend_i = None
        for c in load_stage(0):
            c.wait()
        if n_stages > 1:
            pend_i = load_stage(1)
        for g in range(n_groups):
            s, j = divmod(g, SG)
            slot = s & 1
            if j == 0 and s > 0:
                for c in pend_i:
                    c.wait()
                if s + 1 < n_stages:
                    pend_i = load_stage(s + 1)
            gats = [
                pltpu.async_copy(vals_hbm.at[src_v.at[slot, j, b]],
                                 rows_v.at[b], sem_g)
                for b in range(KB)
            ]
            for c in gats:
                c.wait()
            for b in range(KB):
                pltpu.sync_copy(rows_v.at[b],
                                acc.at[dst_v.at[slot, j, b]], add=True)

    @pl.when(cid == 0)
    def _():
        run(G0, 0)

    @pl.when(cid == 1)
    def _():
        run(G1, G0)

    plsc.subcore_barrier()
    pltpu.sync_copy(acc.at[pl.ds(sid * RPT, RPT)],
                    out_hbm.at[cid, pl.ds(sid * RPT, RPT)])


_segsum = functools.partial(
    pl.kernel,
    mesh=plsc.VectorSubcoreMesh(core_axis_name="c", subcore_axis_name="s"),
    out_type=jax.ShapeDtypeStruct((NC, NPAD, NHID), jnp.float32),
    scratch_types=[
        pltpu.VMEM((2, SG, KB, EL), jnp.int32),
        pltpu.VMEM((2, SG, KB, EL), jnp.int32),
        pltpu.VMEM((KB, EL, NHID), jnp.float32),
        pltpu.VMEM_SHARED((NPAD, NHID), jnp.float32),
        pltpu.SemaphoreType.DMA,
        pltpu.SemaphoreType.DMA,
    ],
    compiler_params=pltpu.CompilerParams(use_tc_tiling_on_sc=False),
)(_segsum_body)


def _mm_body(a_ref, w_ref, o_ref):
    o_ref[...] = jnp.dot(a_ref[...], w_ref[...],
                         preferred_element_type=jnp.float32)


def _proj(a, w):
    return pl.pallas_call(
        _mm_body,
        out_shape=jax.ShapeDtypeStruct((a.shape[0], w.shape[1]), jnp.float32),
    )(a, w)


def _comb_relu_mm_body(p_ref, b_ref, w_ref, o_ref):
    x = jnp.maximum(p_ref[0] + p_ref[1] + b_ref[...], 0.0)
    o_ref[...] = jnp.dot(x, w_ref[...], preferred_element_type=jnp.float32)


def _comb_relu_mm(p, b, w):
    return pl.pallas_call(
        _comb_relu_mm_body,
        out_shape=jax.ShapeDtypeStruct((p.shape[1], w.shape[1]), jnp.float32),
    )(p, b.reshape(1, -1), w)


def _comb_relu_body(p_ref, b_ref, o_ref):
    o_ref[...] = jnp.maximum(p_ref[0] + p_ref[1] + b_ref[...], 0.0)


def _comb_relu(p, b):
    return pl.pallas_call(
        _comb_relu_body,
        out_shape=jax.ShapeDtypeStruct((p.shape[1], p.shape[2]), jnp.float32),
    )(p, b.reshape(1, -1))


def _comb_mm2_body(p_ref, wa_ref, ba_ref, ws_ref, bs_ref, xa_ref, s_ref):
    c = p_ref[0] + p_ref[1]
    xa_ref[...] = jnp.maximum(
        jnp.dot(c, wa_ref[...], preferred_element_type=jnp.float32)
        + ba_ref[...], 0.0)
    s_ref[...] = jnp.maximum(
        jnp.dot(c, ws_ref[...], preferred_element_type=jnp.float32)
        + bs_ref[...], 0.0)


def _comb_mm2(p, wa, ba, ws, bs):
    return pl.pallas_call(
        _comb_mm2_body,
        out_shape=(
            jax.ShapeDtypeStruct((p.shape[1], wa.shape[1]), jnp.float32),
            jax.ShapeDtypeStruct((p.shape[1], ws.shape[1]), jnp.float32),
        ),
    )(p, wa, ba.reshape(1, -1), ws, bs.reshape(1, -1))


def _comb_mm_relu_body(p_ref, w_ref, b_ref, o_ref):
    o_ref[...] = jnp.maximum(
        jnp.dot(p_ref[0] + p_ref[1], w_ref[...],
                preferred_element_type=jnp.float32) + b_ref[...], 0.0)


def _comb_mm_relu(p, w, b):
    return pl.pallas_call(
        _comb_mm_relu_body,
        out_shape=jax.ShapeDtypeStruct((p.shape[1], w.shape[1]), jnp.float32),
    )(p, w, b.reshape(1, -1))


_BM = 128


def _outer_body(a_ref, b_ref, o_ref):
    o_ref[...] = lax.dot_general(
        a_ref[...], b_ref[...], (((1,), (1,)), ((), ())),
        preferred_element_type=jnp.float32)


def _outer(s):
    n = s.shape[0]
    return pl.pallas_call(
        _outer_body,
        grid=(pl.cdiv(n, _BM),),
        in_specs=[
            pl.BlockSpec((_BM, NHID), lambda i: (i, 0)),
            pl.BlockSpec((n, NHID), lambda i: (0, 0)),
        ],
        out_specs=pl.BlockSpec((_BM, n), lambda i: (i, 0)),
        out_shape=jax.ShapeDtypeStruct((n, n), jnp.float32),
    )(s, s)


def kernel(h, edge_index, W_e1, b_e1, W_e2, b_e2, W_a1, b_a1, W_a2, b_a2,
           W_s1, b_s1):
    src = edge_index[0].astype(jnp.int32)
    dst = edge_index[1].astype(jnp.int32)
    pad = EPAD - E
    extra = (NGRP_ALLOC - NGRP) * EG
    src2d = jnp.concatenate(
        [src, jnp.zeros((pad + extra,), jnp.int32)]).reshape(
            NGRP_ALLOC, KB, EL)
    dst2d = jnp.concatenate(
        [dst, jnp.full((pad,), N, jnp.int32),
         jnp.zeros((extra,), jnp.int32)]).reshape(NGRP_ALLOC, KB, EL)
    zeros = jnp.zeros((NPAD, NHID), jnp.float32)

    def segsum(vals):
        out = _segsum(src2d, dst2d, vals, zeros)
        return out[:, :N, :]

    m1 = _proj(h, W_e1)
    p = segsum(m1)
    x1m = _comb_relu_mm(p, b_e1, W_e2)
    q = segsum(x1m)
    x2 = _comb_relu(q, b_e2)
    r = segsum(x2)
    xa, s = _comb_mm2(r, W_a1, b_a1, W_s1, b_s1)
    t = segsum(xa)
    x_hat = _comb_mm_relu(t, W_a2, b_a2)
    struct = _outer(s)
    return (struct, x_hat)


# speedup vs baseline: 2.6424x; 1.7915x over previous
import functools

import jax
import jax.numpy as jnp
from jax import lax
from jax.experimental import pallas as pl
from jax.experimental.pallas import tpu as pltpu
from jax.experimental.pallas import tpu_sc as plsc

N = 10000
NFEAT = 128
NHID = 64
E = 320000

NC = 2
NS = 16
EL = 96
KB = 6
EG = KB * EL
SG = 3
G0 = 18
G1 = 18
GT = G0 + G1
NGRP = NS * GT
NGRP_ALLOC = NGRP + 8
EPAD = NGRP * EG
NPAD = N + 112
RPT = NPAD // NS


def _segsum_body(src_hbm, dst_hbm, vals_hbm, zeros_hbm, out_hbm,
                 src_v, dst_v, rows_v, spvals, acc, sem_g, sem_i):
    cid = lax.axis_index("c")
    sid = lax.axis_index("s")

    pltpu.sync_copy(zeros_hbm.at[pl.ds(sid * RPT, RPT)],
                    acc.at[pl.ds(sid * RPT, RPT)])
    pltpu.sync_copy(vals_hbm.at[pl.ds(sid * RPT, RPT)],
                    spvals.at[pl.ds(sid * RPT, RPT)])
    plsc.subcore_barrier()

    def run(n_groups, off):
        base = sid * GT + off
        n_stages = (n_groups + SG - 1) // SG

        def load_stage(s):
            slot = s & 1
            return [
                pltpu.async_copy(src_hbm.at[pl.ds(base + s * SG, SG)],
                                 src_v.at[slot], sem_i),
                pltpu.async_copy(dst_hbm.at[pl.ds(base + s * SG, SG)],
                                 dst_v.at[slot], sem_i),
            ]

        pend_i = None
        for c in load_stage(0):
            c.wait()
        if n_stages > 1:
            pend_i = load_stage(1)
        for g in range(n_groups):
            s, j = divmod(g, SG)
            slot = s & 1
            if j == 0 and s > 0:
                for c in pend_i:
                    c.wait()
                if s + 1 < n_stages:
                    pend_i = load_stage(s + 1)
            gats = [
                pltpu.async_copy(spvals.at[src_v.at[slot, j, b]],
                                 rows_v.at[b], sem_g)
                for b in range(KB)
            ]
            for c in gats:
                c.wait()
            for b in range(KB):
                pltpu.sync_copy(rows_v.at[b],
                                acc.at[dst_v.at[slot, j, b]], add=True)

    @pl.when(cid == 0)
    def _():
        run(G0, 0)

    @pl.when(cid == 1)
    def _():
        run(G1, G0)

    plsc.subcore_barrier()
    pltpu.sync_copy(acc.at[pl.ds(sid * RPT, RPT)],
                    out_hbm.at[cid, pl.ds(sid * RPT, RPT)])


_segsum = functools.partial(
    pl.kernel,
    mesh=plsc.VectorSubcoreMesh(core_axis_name="c", subcore_axis_name="s"),
    out_type=jax.ShapeDtypeStruct((NC, NPAD, NHID), jnp.float32),
    scratch_types=[
        pltpu.VMEM((2, SG, KB, EL), jnp.int32),
        pltpu.VMEM((2, SG, KB, EL), jnp.int32),
        pltpu.VMEM((KB, EL, NHID), jnp.float32),
        pltpu.VMEM_SHARED((NPAD, NHID), jnp.float32),
        pltpu.VMEM_SHARED((NPAD, NHID), jnp.float32),
        pltpu.SemaphoreType.DMA,
        pltpu.SemaphoreType.DMA,
    ],
    compiler_params=pltpu.CompilerParams(use_tc_tiling_on_sc=False),
)(_segsum_body)


def _mm_body(a_ref, w_ref, o_ref):
    o_ref[...] = jnp.dot(a_ref[...], w_ref[...],
                         preferred_element_type=jnp.float32)


def _proj(a, w):
    return pl.pallas_call(
        _mm_body,
        out_shape=jax.ShapeDtypeStruct((a.shape[0], w.shape[1]), jnp.float32),
    )(a, w)


def _comb_relu_mm_body(p_ref, b_ref, w_ref, o_ref):
    x = jnp.maximum(p_ref[0] + p_ref[1] + b_ref[...], 0.0)
    o_ref[...] = jnp.dot(x, w_ref[...], preferred_element_type=jnp.float32)


def _comb_relu_mm(p, b, w):
    return pl.pallas_call(
        _comb_relu_mm_body,
        out_shape=jax.ShapeDtypeStruct((p.shape[1], w.shape[1]), jnp.float32),
    )(p, b.reshape(1, -1), w)


def _comb_relu_body(p_ref, b_ref, o_ref):
    o_ref[...] = jnp.maximum(p_ref[0] + p_ref[1] + b_ref[...], 0.0)


def _comb_relu(p, b):
    return pl.pallas_call(
        _comb_relu_body,
        out_shape=jax.ShapeDtypeStruct((p.shape[1], p.shape[2]), jnp.float32),
    )(p, b.reshape(1, -1))


def _comb_mm2_body(p_ref, wa_ref, ba_ref, ws_ref, bs_ref, xa_ref, s_ref):
    c = p_ref[0] + p_ref[1]
    xa_ref[...] = jnp.maximum(
        jnp.dot(c, wa_ref[...], preferred_element_type=jnp.float32)
        + ba_ref[...], 0.0)
    s_ref[...] = jnp.maximum(
        jnp.dot(c, ws_ref[...], preferred_element_type=jnp.float32)
        + bs_ref[...], 0.0)


def _comb_mm2(p, wa, ba, ws, bs):
    return pl.pallas_call(
        _comb_mm2_body,
        out_shape=(
            jax.ShapeDtypeStruct((p.shape[1], wa.shape[1]), jnp.float32),
            jax.ShapeDtypeStruct((p.shape[1], ws.shape[1]), jnp.float32),
        ),
    )(p, wa, ba.reshape(1, -1), ws, bs.reshape(1, -1))


def _comb_mm_relu_body(p_ref, w_ref, b_ref, o_ref):
    o_ref[...] = jnp.maximum(
        jnp.dot(p_ref[0] + p_ref[1], w_ref[...],
                preferred_element_type=jnp.float32) + b_ref[...], 0.0)


def _comb_mm_relu(p, w, b):
    return pl.pallas_call(
        _comb_mm_relu_body,
        out_shape=jax.ShapeDtypeStruct((p.shape[1], w.shape[1]), jnp.float32),
    )(p, w, b.reshape(1, -1))


_BM = 128


def _outer_body(a_ref, b_ref, o_ref):
    o_ref[...] = lax.dot_general(
        a_ref[...], b_ref[...], (((1,), (1,)), ((), ())),
        preferred_element_type=jnp.float32)


def _outer(s):
    n = s.shape[0]
    return pl.pallas_call(
        _outer_body,
        grid=(pl.cdiv(n, _BM),),
        in_specs=[
            pl.BlockSpec((_BM, NHID), lambda i: (i, 0)),
            pl.BlockSpec((n, NHID), lambda i: (0, 0)),
        ],
        out_specs=pl.BlockSpec((_BM, n), lambda i: (i, 0)),
        out_shape=jax.ShapeDtypeStruct((n, n), jnp.float32),
    )(s, s)


def kernel(h, edge_index, W_e1, b_e1, W_e2, b_e2, W_a1, b_a1, W_a2, b_a2,
           W_s1, b_s1):
    src = edge_index[0].astype(jnp.int32)
    dst = edge_index[1].astype(jnp.int32)
    pad = EPAD - E
    extra = (NGRP_ALLOC - NGRP) * EG
    src2d = jnp.concatenate(
        [src, jnp.zeros((pad + extra,), jnp.int32)]).reshape(
            NGRP_ALLOC, KB, EL)
    dst2d = jnp.concatenate(
        [dst, jnp.full((pad,), N, jnp.int32),
         jnp.zeros((extra,), jnp.int32)]).reshape(NGRP_ALLOC, KB, EL)
    zeros = jnp.zeros((NPAD, NHID), jnp.float32)

    def segsum(vals):
        vals_p = jnp.concatenate(
            [vals, jnp.zeros((NPAD - N, NHID), jnp.float32)])
        out = _segsum(src2d, dst2d, vals_p, zeros)
        return out[:, :N, :]

    m1 = _proj(h, W_e1)
    p = segsum(m1)
    x1m = _comb_relu_mm(p, b_e1, W_e2)
    q = segsum(x1m)
    x2 = _comb_relu(q, b_e2)
    r = segsum(x2)
    xa, s = _comb_mm2(r, W_a1, b_a1, W_s1, b_s1)
    t = segsum(xa)
    x_hat = _comb_mm_relu(t, W_a2, b_a2)
    struct = _outer(s)
    return (struct, x_hat)


# speedup vs baseline: 2.7519x; 1.0414x over previous
import functools

import jax
import jax.numpy as jnp
from jax import lax
from jax.experimental import pallas as pl
from jax.experimental.pallas import tpu as pltpu
from jax.experimental.pallas import tpu_sc as plsc

N = 10000
NFEAT = 128
NHID = 64
E = 320000

NC = 2
NS = 16
EL = 96
KB = 6
EG = KB * EL
SG = 3
G0 = 20
G1 = 16
GT = G0 + G1
NGRP = NS * GT
NGRP_ALLOC = NGRP + 8
EPAD = NGRP * EG
NPAD = N + 112
RPT = NPAD // NS


def _segsum_body(src_hbm, dst_hbm, vals_hbm, zeros_hbm, out_hbm,
                 src_v, dst_v, rows_v, spvals, acc, sem_g, sem_i):
    cid = lax.axis_index("c")
    sid = lax.axis_index("s")

    pltpu.sync_copy(zeros_hbm.at[pl.ds(sid * RPT, RPT)],
                    acc.at[pl.ds(sid * RPT, RPT)])
    pltpu.sync_copy(vals_hbm.at[pl.ds(sid * RPT, RPT)],
                    spvals.at[pl.ds(sid * RPT, RPT)])
    plsc.subcore_barrier()

    def run(n_groups, off):
        base = sid * GT + off
        n_stages = (n_groups + SG - 1) // SG

        def load_stage(s):
            slot = s & 1
            return [
                pltpu.async_copy(src_hbm.at[pl.ds(base + s * SG, SG)],
                                 src_v.at[slot], sem_i),
                pltpu.async_copy(dst_hbm.at[pl.ds(base + s * SG, SG)],
                                 dst_v.at[slot], sem_i),
            ]

        pend_i = None
        for c in load_stage(0):
            c.wait()
        if n_stages > 1:
            pend_i = load_stage(1)
        for g in range(n_groups):
            s, j = divmod(g, SG)
            slot = s & 1
            if j == 0 and s > 0:
                for c in pend_i:
                    c.wait()
                if s + 1 < n_stages:
                    pend_i = load_stage(s + 1)
            gats = [
                pltpu.async_copy(spvals.at[src_v.at[slot, j, b]],
                                 rows_v.at[b], sem_g)
                for b in range(KB)
            ]
            for c in gats:
                c.wait()
            for b in range(KB):
                pltpu.sync_copy(rows_v.at[b],
                                acc.at[dst_v.at[slot, j, b]], add=True)

    @pl.when(cid == 0)
    def _():
        run(G0, 0)

    @pl.when(cid == 1)
    def _():
        run(G1, G0)

    plsc.subcore_barrier()
    pltpu.sync_copy(acc.at[pl.ds(sid * RPT, RPT)],
                    out_hbm.at[cid, pl.ds(sid * RPT, RPT)])


_segsum = functools.partial(
    pl.kernel,
    mesh=plsc.VectorSubcoreMesh(core_axis_name="c", subcore_axis_name="s"),
    out_type=jax.ShapeDtypeStruct((NC, NPAD, NHID), jnp.float32),
    scratch_types=[
        pltpu.VMEM((2, SG, KB, EL), jnp.int32),
        pltpu.VMEM((2, SG, KB, EL), jnp.int32),
        pltpu.VMEM((KB, EL, NHID), jnp.float32),
        pltpu.VMEM_SHARED((NPAD, NHID), jnp.float32),
        pltpu.VMEM_SHARED((NPAD, NHID), jnp.float32),
        pltpu.SemaphoreType.DMA,
        pltpu.SemaphoreType.DMA,
    ],
    compiler_params=pltpu.CompilerParams(use_tc_tiling_on_sc=False),
)(_segsum_body)


def _mm_body(a_ref, w_ref, o_ref):
    o_ref[...] = jnp.dot(a_ref[...], w_ref[...],
                         preferred_element_type=jnp.float32)


def _proj(a, w):
    return pl.pallas_call(
        _mm_body,
        out_shape=jax.ShapeDtypeStruct((a.shape[0], w.shape[1]), jnp.float32),
    )(a, w)


def _comb_relu_mm_body(p_ref, b_ref, w_ref, o_ref):
    x = jnp.maximum(p_ref[0] + p_ref[1] + b_ref[...], 0.0)
    o_ref[...] = jnp.dot(x, w_ref[...], preferred_element_type=jnp.float32)


def _comb_relu_mm(p, b, w):
    return pl.pallas_call(
        _comb_relu_mm_body,
        out_shape=jax.ShapeDtypeStruct((p.shape[1], w.shape[1]), jnp.float32),
    )(p, b.reshape(1, -1), w)


def _comb_relu_body(p_ref, b_ref, o_ref):
    o_ref[...] = jnp.maximum(p_ref[0] + p_ref[1] + b_ref[...], 0.0)


def _comb_relu(p, b):
    return pl.pallas_call(
        _comb_relu_body,
        out_shape=jax.ShapeDtypeStruct((p.shape[1], p.shape[2]), jnp.float32),
    )(p, b.reshape(1, -1))


def _comb_mm2_body(p_ref, wa_ref, ba_ref, ws_ref, bs_ref, xa_ref, s_ref):
    c = p_ref[0] + p_ref[1]
    xa_ref[...] = jnp.maximum(
        jnp.dot(c, wa_ref[...], preferred_element_type=jnp.float32)
        + ba_ref[...], 0.0)
    s_ref[...] = jnp.maximum(
        jnp.dot(c, ws_ref[...], preferred_element_type=jnp.float32)
        + bs_ref[...], 0.0)


def _comb_mm2(p, wa, ba, ws, bs):
    return pl.pallas_call(
        _comb_mm2_body,
        out_shape=(
            jax.ShapeDtypeStruct((p.shape[1], wa.shape[1]), jnp.float32),
            jax.ShapeDtypeStruct((p.shape[1], ws.shape[1]), jnp.float32),
        ),
    )(p, wa, ba.reshape(1, -1), ws, bs.reshape(1, -1))


def _comb_mm_relu_body(p_ref, w_ref, b_ref, o_ref):
    o_ref[...] = jnp.maximum(
        jnp.dot(p_ref[0] + p_ref[1], w_ref[...],
                preferred_element_type=jnp.float32) + b_ref[...], 0.0)


def _comb_mm_relu(p, w, b):
    return pl.pallas_call(
        _comb_mm_relu_body,
        out_shape=jax.ShapeDtypeStruct((p.shape[1], w.shape[1]), jnp.float32),
    )(p, w, b.reshape(1, -1))


_BM = 128


def _outer_body(a_ref, b_ref, o_ref):
    o_ref[...] = lax.dot_general(
        a_ref[...], b_ref[...], (((1,), (1,)), ((), ())),
        preferred_element_type=jnp.float32)


def _outer(s):
    n = s.shape[0]
    return pl.pallas_call(
        _outer_body,
        grid=(pl.cdiv(n, _BM),),
        in_specs=[
            pl.BlockSpec((_BM, NHID), lambda i: (i, 0)),
            pl.BlockSpec((n, NHID), lambda i: (0, 0)),
        ],
        out_specs=pl.BlockSpec((_BM, n), lambda i: (i, 0)),
        out_shape=jax.ShapeDtypeStruct((n, n), jnp.float32),
    )(s, s)


def kernel(h, edge_index, W_e1, b_e1, W_e2, b_e2, W_a1, b_a1, W_a2, b_a2,
           W_s1, b_s1):
    src = edge_index[0].astype(jnp.int32)
    dst = edge_index[1].astype(jnp.int32)
    pad = EPAD - E
    extra = (NGRP_ALLOC - NGRP) * EG
    src2d = jnp.concatenate(
        [src, jnp.zeros((pad + extra,), jnp.int32)]).reshape(
            NGRP_ALLOC, KB, EL)
    dst2d = jnp.concatenate(
        [dst, jnp.full((pad,), N, jnp.int32),
         jnp.zeros((extra,), jnp.int32)]).reshape(NGRP_ALLOC, KB, EL)
    zeros = jnp.zeros((NPAD, NHID), jnp.float32)

    def segsum(vals):
        vals_p = jnp.concatenate(
            [vals, jnp.zeros((NPAD - N, NHID), jnp.float32)])
        out = _segsum(src2d, dst2d, vals_p, zeros)
        return out[:, :N, :]

    m1 = _proj(h, W_e1)
    p = segsum(m1)
    x1m = _comb_relu_mm(p, b_e1, W_e2)
    q = segsum(x1m)
    x2 = _comb_relu(q, b_e2)
    r = segsum(x2)
    xa, s = _comb_mm2(r, W_a1, b_a1, W_s1, b_s1)
    t = segsum(xa)
    x_hat = _comb_mm_relu(t, W_a2, b_a2)
    struct = _outer(s)
    return (struct, x_hat)
